# baseline (speedup 1.0000x reference)
import functools

import jax
import jax.numpy as jnp
from jax import lax
from jax.experimental import pallas as pl
from jax.experimental.pallas import tpu as pltpu
from jax.experimental.pallas import tpu_sc as plsc

KERNEL = 512
NUM_WORKERS = 32
CHUNK = 2048
NBUF = 3


@functools.cache
def _make_sc_gather(n_pairs, row_d):
    per_w = n_pairs // NUM_WORKERS
    n_iter = per_w // CHUNK
    assert per_w % CHUNK == 0
    mesh = plsc.VectorSubcoreMesh(core_axis_name="c", subcore_axis_name="s")
    nbuf = min(NBUF, n_iter)

    @functools.partial(
        pl.kernel,
        mesh=mesh,
        out_type=jax.ShapeDtypeStruct((n_pairs, row_d), jnp.float32),
        scratch_types=(
            [pltpu.VMEM((n_iter, CHUNK), jnp.int32)]
            + [pltpu.VMEM((CHUNK, row_d), jnp.float32)] * nbuf
            + [pltpu.SemaphoreType.DMA] * (2 * nbuf)
        ),
        compiler_params=pltpu.CompilerParams(use_tc_tiling_on_sc=False),
    )
    def gather_k(idx_hbm, table_hbm, out_hbm, idx_v, *bufs_sems):
        rows = bufs_sems[:nbuf]
        sg = bufs_sems[nbuf:2 * nbuf]
        ss = bufs_sems[2 * nbuf:]
        wid = lax.axis_index("s") * 2 + lax.axis_index("c")
        base = wid * per_w
        pltpu.sync_copy(idx_hbm.at[wid], idx_v)

        gh = {}
        sh = {}
        for j in range(nbuf):
            gh[j] = pltpu.async_copy(
                table_hbm.at[idx_v.at[j]], rows[j % nbuf], sg[j % nbuf])
        for j in range(n_iter):
            gh[j].wait()
            sh[j] = pltpu.async_copy(
                rows[j % nbuf],
                out_hbm.at[pl.ds(base + j * CHUNK, CHUNK)],
                ss[j % nbuf])
            nxt = j + nbuf
            if nxt < n_iter:
                sh[j].wait()
                gh[nxt] = pltpu.async_copy(
                    table_hbm.at[idx_v.at[nxt]], rows[j % nbuf], sg[j % nbuf])
        for j in range(max(0, n_iter - nbuf), n_iter):
            sh[j].wait()

    return gather_k


def _tc_body(e_ref, wf_ref, bias_ref, emb_ref, out_ref, wv_s, lens_s):
    n_b = pl.num_programs(0)
    bidx = pl.program_id(0)
    t = e_ref.shape[0]
    c = wf_ref.shape[1]
    y = jnp.dot(e_ref[...], wf_ref[...], preferred_element_type=jnp.float32)
    y = jnp.maximum(y + bias_ref[...], 0.0)
    wv = jnp.max(y, axis=0, keepdims=True)
    iota_t = lax.broadcasted_iota(jnp.int32, (t, c), 0)
    tw = jnp.min(jnp.where(y == wv, iota_t, t), axis=0, keepdims=True)
    occ = jnp.any(iota_t == tw, axis=1, keepdims=True)
    n_unique = jnp.sum(occ.astype(jnp.float32))
    has_zero = jnp.max(jnp.where(tw == 0, 1.0, 0.0))
    lens = 2.0 * n_unique - has_zero
    wv_s[pl.ds(bidx, 1), :] = wv
    lens_s[pl.ds(bidx, 1), :] = jnp.full((1, c), lens, jnp.float32)

    @pl.when(bidx == n_b - 1)
    def _():
        pad_e = jnp.tile(emb_ref[0:1, :], (1, KERNEL))
        pad_y = jnp.dot(pad_e, wf_ref[...],
                        preferred_element_type=jnp.float32)
        pad_act = jnp.maximum(pad_y + bias_ref[...], 0.0)
        lens_all = lens_s[...]
        maxlen = jnp.maximum(jnp.max(lens_all), 1.0)
        has_pad = lens_all < maxlen
        out_ref[...] = jnp.maximum(wv_s[...],
                                   jnp.where(has_pad, pad_act, -1.0))


def kernel(x, emb, w, b):
    batch, seq_len = x.shape
    out_ch, emb_d, k = w.shape
    vocab = emb.shape[0]
    assert k == KERNEL and seq_len % KERNEL == 0
    t = seq_len // KERNEL
    kd = KERNEL * emb_d

    n_pairs = batch * seq_len // 2
    xp = x.reshape(-1, 2)
    per_w = n_pairs // NUM_WORKERS
    xf2 = (xp[:, 0] * vocab + xp[:, 1]).reshape(
        NUM_WORKERS, per_w // CHUNK, CHUNK)
    table = jnp.concatenate(
        [jnp.repeat(emb, vocab, axis=0), jnp.tile(emb, (vocab, 1))], axis=1)

    e_rows = _make_sc_gather(n_pairs, 2 * emb_d)(xf2, table)
    e2 = e_rows.reshape(batch * t, kd)

    wf = w.transpose(2, 1, 0).reshape(kd, out_ch)
    bias = b.reshape(1, out_ch)

    return pl.pallas_call(
        _tc_body,
        grid=(batch,),
        in_specs=[
            pl.BlockSpec((t, kd), lambda i: (i, 0)),
            pl.BlockSpec((kd, out_ch), lambda i: (0, 0)),
            pl.BlockSpec((1, out_ch), lambda i: (0, 0)),
            pl.BlockSpec((8, emb_d), lambda i: (0, 0)),
        ],
        out_specs=pl.BlockSpec((batch, out_ch), lambda i: (0, 0)),
        out_shape=jax.ShapeDtypeStruct((batch, out_ch), jnp.float32),
        scratch_shapes=[
            pltpu.VMEM((batch, out_ch), jnp.float32),
            pltpu.VMEM((batch, out_ch), jnp.float32),
        ],
        compiler_params=pltpu.CompilerParams(
            dimension_semantics=("arbitrary",)),
    )(e2, wf, bias, emb)


# speedup vs baseline: 3.3435x; 3.3435x over previous
import functools

import jax
import jax.numpy as jnp
from jax import lax
from jax.experimental import pallas as pl
from jax.experimental.pallas import tpu as pltpu
from jax.experimental.pallas import tpu_sc as plsc

KERNEL = 512
NUM_WORKERS = 32
LANES = 16


@functools.cache
def _make_sc_gather(n_tok, table_words, emb_d):
    n_win = n_tok // KERNEL
    per_w_win = n_win // NUM_WORKERS
    per_w_tok = per_w_win * KERNEL
    grp = KERNEL // LANES
    assert n_win % NUM_WORKERS == 0
    mesh = plsc.VectorSubcoreMesh(core_axis_name="c", subcore_axis_name="s")

    @functools.partial(
        pl.kernel,
        mesh=mesh,
        out_type=jax.ShapeDtypeStruct((n_win, emb_d, KERNEL), jnp.float32),
        scratch_types=(
            pltpu.VMEM((per_w_tok,), jnp.int32),
            pltpu.VMEM((table_words,), jnp.float32),
            pltpu.VMEM((2, emb_d, KERNEL), jnp.float32),
            pltpu.SemaphoreType.DMA,
            pltpu.SemaphoreType.DMA,
        ),
        compiler_params=pltpu.CompilerParams(
            use_tc_tiling_on_sc=False, needs_layout_passes=False),
    )
    def gather_k(idx_hbm, table_hbm, out_hbm, idx_v, table_v, obuf, s0, s1):
        wid = lax.axis_index("s") * 2 + lax.axis_index("c")
        pltpu.sync_copy(idx_hbm.at[wid], idx_v)
        pltpu.sync_copy(table_hbm, table_v)
        row0 = wid * per_w_win
        sems = (s0, s1)

        def process(w, buf):
            base = w * KERNEL
            for g in range(grp):
                tok8 = idx_v[pl.ds(base + g * LANES, LANES)] * emb_d
                for d in range(emb_d):
                    v = plsc.load_gather(table_v, [tok8 + d])
                    obuf[buf, d, pl.ds(g * LANES, LANES)] = v

        for b in range(2):
            process(b, b)
            pltpu.async_copy(obuf.at[b], out_hbm.at[row0 + b], sems[b])

        def body(i, carry):
            w0 = i * 2
            for b in range(2):
                pltpu.make_async_copy(
                    obuf.at[b], out_hbm.at[row0], sems[b]).wait()
                process(w0 + b, b)
                pltpu.async_copy(
                    obuf.at[b], out_hbm.at[row0 + w0 + b], sems[b])
            return carry

        lax.fori_loop(1, per_w_win // 2, body, 0)
        for b in range(2):
            pltpu.make_async_copy(obuf.at[b], out_hbm.at[row0], sems[b]).wait()

    return gather_k


def _tc_body(e_ref, wf_ref, bias_ref, pad_ref, out_ref, wv_s, lens_s):
    n_b = pl.num_programs(0)
    bidx = pl.program_id(0)
    t = e_ref.shape[0]
    c = wf_ref.shape[1]
    y = jnp.dot(e_ref[...], wf_ref[...], preferred_element_type=jnp.float32)
    y = jnp.maximum(y + bias_ref[...], 0.0)
    wv = jnp.max(y, axis=0, keepdims=True)
    iota_t = lax.broadcasted_iota(jnp.int32, (t, c), 0)
    tw = jnp.min(jnp.where(y == wv, iota_t, t), axis=0, keepdims=True)
    occ = jnp.any(iota_t == tw, axis=1, keepdims=True)
    n_unique = jnp.sum(occ.astype(jnp.float32))
    has_zero = jnp.max(jnp.where(tw == 0, 1.0, 0.0))
    lens = 2.0 * n_unique - has_zero
    wv_s[pl.ds(bidx, 1), :] = wv
    lens_s[pl.ds(bidx, 1), :] = jnp.full((1, c), lens, jnp.float32)

    @pl.when(bidx == n_b - 1)
    def _():
        pad_y = jnp.dot(pad_ref[...], wf_ref[...],
                        preferred_element_type=jnp.float32)
        pad_act = jnp.maximum(pad_y + bias_ref[...], 0.0)
        lens_all = lens_s[...]
        maxlen = jnp.maximum(jnp.max(lens_all), 1.0)
        has_pad = lens_all < maxlen
        out_ref[...] = jnp.maximum(wv_s[...],
                                   jnp.where(has_pad, pad_act, -1.0))


def kernel(x, emb, w, b):
    batch, seq_len = x.shape
    out_ch, emb_d, k = w.shape
    vocab = emb.shape[0]
    assert k == KERNEL and seq_len % KERNEL == 0
    t = seq_len // KERNEL
    kd = KERNEL * emb_d
    n_tok = batch * seq_len

    xw = x.reshape(NUM_WORKERS, n_tok // NUM_WORKERS)
    e_rows = _make_sc_gather(n_tok, vocab * emb_d, emb_d)(xw, emb.reshape(-1))
    e2 = e_rows.reshape(batch * t, kd)

    wf = w.transpose(1, 2, 0).reshape(kd, out_ch)
    bias = b.reshape(1, out_ch)
    pad_row = jnp.repeat(emb[0:1, :], KERNEL, axis=1)

    return pl.pallas_call(
        _tc_body,
        grid=(batch,),
        in_specs=[
            pl.BlockSpec((t, kd), lambda i: (i, 0)),
            pl.BlockSpec((kd, out_ch), lambda i: (0, 0)),
            pl.BlockSpec((1, out_ch), lambda i: (0, 0)),
            pl.BlockSpec((1, kd), lambda i: (0, 0)),
        ],
        out_specs=pl.BlockSpec((batch, out_ch), lambda i: (0, 0)),
        out_shape=jax.ShapeDtypeStruct((batch, out_ch), jnp.float32),
        scratch_shapes=[
            pltpu.VMEM((batch, out_ch), jnp.float32),
            pltpu.VMEM((batch, out_ch), jnp.float32),
        ],
        compiler_params=pltpu.CompilerParams(
            dimension_semantics=("arbitrary",)),
    )(e2, wf, bias, pad_row)


# speedup vs baseline: 7.4078x; 2.2156x over previous
import functools

import jax
import jax.numpy as jnp
from jax import lax
from jax.experimental import pallas as pl
from jax.experimental.pallas import tpu as pltpu
from jax.experimental.pallas import tpu_sc as plsc

KERNEL = 512
NUM_WORKERS = 32
LANES = 16


@functools.cache
def _make_sc_gather(n_tok, vocab, emb_d):
    n_win = n_tok // KERNEL
    per_w_win = n_win // NUM_WORKERS
    per_w_tok = per_w_win * KERNEL
    grp = KERNEL // LANES
    assert n_win % NUM_WORKERS == 0
    mesh = plsc.VectorSubcoreMesh(core_axis_name="c", subcore_axis_name="s")

    @functools.partial(
        pl.kernel,
        mesh=mesh,
        out_type=jax.ShapeDtypeStruct((n_win, emb_d, KERNEL), jnp.float32),
        scratch_types=(
            pltpu.VMEM((per_w_tok,), jnp.int32),
            pltpu.VMEM((emb_d, vocab), jnp.float32),
            pltpu.VMEM((2, emb_d, KERNEL), jnp.float32),
            pltpu.SemaphoreType.DMA,
            pltpu.SemaphoreType.DMA,
        ),
        compiler_params=pltpu.CompilerParams(
            use_tc_tiling_on_sc=False, needs_layout_passes=False),
    )
    def gather_k(idx_hbm, table_hbm, out_hbm, idx_v, table_v, obuf, s0, s1):
        wid = lax.axis_index("s") * 2 + lax.axis_index("c")
        pltpu.sync_copy(idx_hbm.at[wid], idx_v)
        pltpu.sync_copy(table_hbm, table_v)
        row0 = wid * per_w_win
        sems = (s0, s1)

        def process(w, buf):
            base = w * KERNEL

            @plsc.parallel_loop(0, grp, 1, unroll=4)
            def _(g):
                off = g * LANES
                tok = idx_v[pl.ds(base + off, LANES)]
                for d in range(emb_d):
                    v = plsc.load_gather(table_v.at[d], [tok])
                    obuf[buf, d, pl.ds(off, LANES)] = v

        for b in range(2):
            process(b, b)
            pltpu.async_copy(obuf.at[b], out_hbm.at[row0 + b], sems[b])

        def body(i, carry):
            w0 = i * 2
            for b in range(2):
                pltpu.make_async_copy(
                    obuf.at[b], out_hbm.at[row0], sems[b]).wait()
                process(w0 + b, b)
                pltpu.async_copy(
                    obuf.at[b], out_hbm.at[row0 + w0 + b], sems[b])
            return carry

        lax.fori_loop(1, per_w_win // 2, body, 0)
        for b in range(2):
            pltpu.make_async_copy(obuf.at[b], out_hbm.at[row0], sems[b]).wait()

    return gather_k


def _tc_body(e_ref, wf_ref, bias_ref, pad_ref, out_ref, wv_s, lens_s):
    n_b = pl.num_programs(0)
    bidx = pl.program_id(0)
    t = e_ref.shape[0]
    c = wf_ref.shape[1]
    y = jnp.dot(e_ref[...], wf_ref[...], preferred_element_type=jnp.float32)
    y = jnp.maximum(y + bias_ref[...], 0.0)
    wv = jnp.max(y, axis=0, keepdims=True)
    iota_t = lax.broadcasted_iota(jnp.int32, (t, c), 0)
    tw = jnp.min(jnp.where(y == wv, iota_t, t), axis=0, keepdims=True)
    occ = jnp.any(iota_t == tw, axis=1, keepdims=True)
    n_unique = jnp.sum(occ.astype(jnp.float32))
    has_zero = jnp.max(jnp.where(tw == 0, 1.0, 0.0))
    lens = 2.0 * n_unique - has_zero
    wv_s[pl.ds(bidx, 1), :] = wv
    lens_s[pl.ds(bidx, 1), :] = jnp.full((1, c), lens, jnp.float32)

    @pl.when(bidx == n_b - 1)
    def _():
        pad_y = jnp.dot(pad_ref[...], wf_ref[...],
                        preferred_element_type=jnp.float32)
        pad_act = jnp.maximum(pad_y + bias_ref[...], 0.0)
        lens_all = lens_s[...]
        maxlen = jnp.maximum(jnp.max(lens_all), 1.0)
        has_pad = lens_all < maxlen
        out_ref[...] = jnp.maximum(wv_s[...],
                                   jnp.where(has_pad, pad_act, -1.0))


def kernel(x, emb, w, b):
    batch, seq_len = x.shape
    out_ch, emb_d, k = w.shape
    vocab = emb.shape[0]
    assert k == KERNEL and seq_len % KERNEL == 0
    t = seq_len // KERNEL
    kd = KERNEL * emb_d
    n_tok = batch * seq_len

    xw = x.reshape(NUM_WORKERS, n_tok // NUM_WORKERS)
    e_rows = _make_sc_gather(n_tok, vocab, emb_d)(xw, emb.T)
    e2 = e_rows.reshape(batch * t, kd)

    wf = w.transpose(1, 2, 0).reshape(kd, out_ch)
    bias = b.reshape(1, out_ch)
    pad_row = jnp.repeat(emb[0:1, :], KERNEL, axis=1)

    return pl.pallas_call(
        _tc_body,
        grid=(batch,),
        in_specs=[
            pl.BlockSpec((t, kd), lambda i: (i, 0)),
            pl.BlockSpec((kd, out_ch), lambda i: (0, 0)),
            pl.BlockSpec((1, out_ch), lambda i: (0, 0)),
            pl.BlockSpec((1, kd), lambda i: (0, 0)),
        ],
        out_specs=pl.BlockSpec((batch, out_ch), lambda i: (0, 0)),
        out_shape=jax.ShapeDtypeStruct((batch, out_ch), jnp.float32),
        scratch_shapes=[
            pltpu.VMEM((batch, out_ch), jnp.float32),
            pltpu.VMEM((batch, out_ch), jnp.float32),
        ],
        compiler_params=pltpu.CompilerParams(
            dimension_semantics=("arbitrary",)),
    )(e2, wf, bias, pad_row)
